# baseline (speedup 1.0000x reference)
import functools

import jax
import jax.numpy as jnp
from jax import lax
from jax.experimental import pallas as pl
from jax.experimental.pallas import tpu as pltpu
from jax.experimental.pallas import tpu_sc as plsc

N = 50000
M = 2048
D = 3
F = 3
K = 3
NPAD = 50176
BQ = 32

NR = N // 8
XW = 8 * F

NC = 2
NS = 16
NW = NC * NS
QW = M // NW
RW = QW * K
CH = RW // 2
FP = 16


def _knn_body(py_ref, pxt_ref, xr_ref, idx_ref, w_ref, wb_ref, xp_ref,
              cv_ref, ci_ref):
    xr = xr_ref[...]
    zero13 = jnp.zeros((NR, FP - F), jnp.float32)
    pieces = []
    for j in range(8):
        pieces.append(xr[:, 3 * j:3 * j + 3])
        pieces.append(zero13)
    xp_ref[...] = jnp.concatenate(pieces, axis=1)

    lane = lax.broadcasted_iota(jnp.int32, (1, 128), 1).astype(jnp.float32)
    inf = jnp.float32(jnp.inf)

    def qblock(qb, _):
        py = py_ref[pl.ds(qb * BQ, BQ), :]
        py0 = py[:, 0:1]
        py1 = py[:, 1:2]
        py2 = py[:, 2:3]
        carry = (
            jnp.full((BQ, 128), inf), jnp.full((BQ, 128), inf),
            jnp.full((BQ, 128), inf),
            jnp.zeros((BQ, 128)), jnp.zeros((BQ, 128)), jnp.zeros((BQ, 128)),
        )

        def fold(t, carry):
            m1, m2, m3, i1, i2, i3 = carry
            px = pxt_ref[:, t * 128:(t + 1) * 128]
            c0 = py0 - px[0:1, :]
            c1 = py1 - px[1:2, :]
            c2 = py2 - px[2:3, :]
            c = (c0 * c0 + c1 * c1) + c2 * c2
            icb = jnp.broadcast_to(jnp.float32(t * 128) + lane, (BQ, 128))
            p1 = c < m1
            n_m1 = jnp.minimum(c, m1)
            s1 = jnp.maximum(c, m1)
            n_i1 = jnp.where(p1, icb, i1)
            si1 = jnp.where(p1, i1, icb)
            p2 = s1 < m2
            n_m2 = jnp.minimum(s1, m2)
            s2 = jnp.maximum(s1, m2)
            n_i2 = jnp.where(p2, si1, i2)
            si2 = jnp.where(p2, i2, si1)
            p3 = s2 < m3
            n_m3 = jnp.minimum(s2, m3)
            n_i3 = jnp.where(p3, si2, i3)
            return (n_m1, n_m2, n_m3, n_i1, n_i2, n_i3)

        for t in range(NPAD // 128):
            carry = fold(t, carry)
        m1, m2, m3, i1, i2, i3 = carry
        cv_ref[pl.ds(qb * BQ, BQ), :] = jnp.concatenate([m1, m2, m3], axis=1)
        ci_ref[pl.ds(qb * BQ, BQ), :] = jnp.concatenate([i1, i2, i3], axis=1)
        return 0

    lax.fori_loop(0, M // BQ, qblock, 0)

    C = cv_ref[...]
    I = ci_ref[...]
    pos = lax.broadcasted_iota(jnp.int32, (M, 384), 1).astype(jnp.float32)
    big = jnp.float32(1e9)
    iv_cols = []
    w_cols = []
    for j in range(K):
        m = jnp.min(C, axis=1, keepdims=True)
        pm = jnp.min(jnp.where(C == m, pos, big), axis=1, keepdims=True)
        iv = jnp.min(jnp.where(pos == pm, I, big), axis=1, keepdims=True)
        iv_cols.append(iv.astype(jnp.int32))
        w_cols.append(1.0 / jnp.clip(m, 1e-16, None))
        if j < K - 1:
            C = jnp.where(pos == pm, inf, C)
    idx_ref[...] = jnp.concatenate(iv_cols, axis=1)
    w_ref[...] = jnp.concatenate(w_cols, axis=1)
    wb_ref[...] = jnp.concatenate(
        [jnp.broadcast_to(w_cols[j], (M, FP)) for j in range(K)], axis=0)


_knn = pl.pallas_call(
    _knn_body,
    in_specs=[
        pl.BlockSpec((M, D), lambda: (0, 0)),
        pl.BlockSpec((D, NPAD), lambda: (0, 0)),
        pl.BlockSpec((NR, XW), lambda: (0, 0)),
    ],
    out_specs=[
        pl.BlockSpec((M, K), lambda: (0, 0)),
        pl.BlockSpec((M, K), lambda: (0, 0)),
        pl.BlockSpec((K * M, FP), lambda: (0, 0)),
        pl.BlockSpec((NR, 8 * FP), lambda: (0, 0)),
    ],
    out_shape=[
        jax.ShapeDtypeStruct((M, K), jnp.int32),
        jax.ShapeDtypeStruct((M, K), jnp.float32),
        jax.ShapeDtypeStruct((K * M, FP), jnp.float32),
        jax.ShapeDtypeStruct((NR, 8 * FP), jnp.float32),
    ],
    scratch_shapes=[
        pltpu.VMEM((M, 3 * 128), jnp.float32),
        pltpu.VMEM((M, 3 * 128), jnp.float32),
    ],
)


def _combine_body(xp_hbm, idx_hbm, wb_hbm, y_hbm, idx_v, rows_v,
                  wb0_v, wb1_v, wb2_v, y_v, sem):
    wid = lax.axis_index("s") * NC + lax.axis_index("c")
    pltpu.sync_copy(idx_hbm.at[wid], idx_v)
    c0 = pltpu.async_copy(xp_hbm.at[idx_v.at[0]], rows_v.at[pl.ds(0, CH)], sem)
    c1 = pltpu.async_copy(xp_hbm.at[idx_v.at[1]], rows_v.at[pl.ds(CH, CH)], sem)
    pltpu.sync_copy(wb_hbm.at[pl.ds(wid * QW, QW)], wb0_v)
    pltpu.sync_copy(wb_hbm.at[pl.ds(M + wid * QW, QW)], wb1_v)
    pltpu.sync_copy(wb_hbm.at[pl.ds(2 * M + wid * QW, QW)], wb2_v)
    c0.wait()
    c1.wait()

    @pl.loop(0, QW)
    def _(q):
        g0 = rows_v[3 * q]
        g1 = rows_v[3 * q + 1]
        g2 = rows_v[3 * q + 2]
        w0 = wb0_v[q]
        w1 = wb1_v[q]
        w2 = wb2_v[q]
        y_v[q] = (g0 * w0 + g1 * w1 + g2 * w2) / ((w0 + w1) + w2)

    pltpu.sync_copy(y_v, y_hbm.at[pl.ds(wid * QW, QW)])


@functools.cache
def _make_combine():
    return pl.kernel(
        _combine_body,
        mesh=plsc.VectorSubcoreMesh(core_axis_name="c", subcore_axis_name="s",
                                    num_cores=NC, num_subcores=NS),
        out_type=jax.ShapeDtypeStruct((M, FP), jnp.float32),
        scratch_types=[
            pltpu.VMEM((2, CH), jnp.int32),
            pltpu.VMEM((RW, FP), jnp.float32),
            pltpu.VMEM((QW, FP), jnp.float32),
            pltpu.VMEM((QW, FP), jnp.float32),
            pltpu.VMEM((QW, FP), jnp.float32),
            pltpu.VMEM((QW, FP), jnp.float32),
            pltpu.SemaphoreType.DMA,
        ],
        compiler_params=pltpu.CompilerParams(use_tc_tiling_on_sc=False),
    )


def kernel(x, pos_x, pos_y, k):
    del k
    pxt = jnp.concatenate(
        [pos_x.T, jnp.full((D, NPAD - N), 1e6, jnp.float32)], axis=1)
    xr = x.reshape(NR, XW)
    idx, w, wb, xp = _knn(pos_y, pxt, xr)
    x_idx = idx.reshape(-1)
    y_idx = jnp.repeat(jnp.arange(M, dtype=jnp.int32), K)
    weights = w.reshape(-1, 1)
    y16 = _make_combine()(xp.reshape(N, FP), x_idx.reshape(NW, 2, CH), wb)
    return (y16[:, :F], x_idx, y_idx, weights)


# speedup vs baseline: 1.0310x; 1.0310x over previous
import functools

import jax
import jax.numpy as jnp
from jax import lax
from jax.experimental import pallas as pl
from jax.experimental.pallas import tpu as pltpu
from jax.experimental.pallas import tpu_sc as plsc

N = 50000
M = 2048
D = 3
F = 3
K = 3
NPAD = 50176
BQ = 16

NR = N // 8
XW = 8 * F

NC = 2
NS = 16
NW = NC * NS
QW = M // NW
RW = QW * K
CH = RW // 2
FP = 16


def _knn_body(py_ref, pxt_ref, xr_ref, idx_ref, w_ref, wb_ref, xp_ref,
              cv_ref, ci_ref):
    xr = xr_ref[...]
    zero13 = jnp.zeros((NR, FP - F), jnp.float32)
    pieces = []
    for j in range(8):
        pieces.append(xr[:, 3 * j:3 * j + 3])
        pieces.append(zero13)
    xp_ref[...] = jnp.concatenate(pieces, axis=1)

    lane = lax.broadcasted_iota(jnp.int32, (1, 128), 1).astype(jnp.float32)
    inf = jnp.float32(jnp.inf)

    def qblock(qb2, _):
      for sub in range(2):
        qb = qb2 * 2 + sub
        py = py_ref[pl.ds(qb * BQ, BQ), :]
        py0 = py[:, 0:1]
        py1 = py[:, 1:2]
        py2 = py[:, 2:3]
        carry = (
            jnp.full((BQ, 128), inf), jnp.full((BQ, 128), inf),
            jnp.full((BQ, 128), inf),
            jnp.zeros((BQ, 128)), jnp.zeros((BQ, 128)), jnp.zeros((BQ, 128)),
        )

        def fold(t, carry):
            m1, m2, m3, i1, i2, i3 = carry
            px = pxt_ref[:, t * 128:(t + 1) * 128]
            c0 = py0 - px[0:1, :]
            c1 = py1 - px[1:2, :]
            c2 = py2 - px[2:3, :]
            c = (c0 * c0 + c1 * c1) + c2 * c2
            icb = jnp.broadcast_to(jnp.float32(t * 128) + lane, (BQ, 128))
            p1 = c < m1
            n_m1 = jnp.minimum(c, m1)
            s1 = jnp.maximum(c, m1)
            n_i1 = jnp.where(p1, icb, i1)
            si1 = jnp.where(p1, i1, icb)
            p2 = s1 < m2
            n_m2 = jnp.minimum(s1, m2)
            s2 = jnp.maximum(s1, m2)
            n_i2 = jnp.where(p2, si1, i2)
            si2 = jnp.where(p2, i2, si1)
            p3 = s2 < m3
            n_m3 = jnp.minimum(s2, m3)
            n_i3 = jnp.where(p3, si2, i3)
            return (n_m1, n_m2, n_m3, n_i1, n_i2, n_i3)

        for t in range(NPAD // 128):
            carry = fold(t, carry)
        m1, m2, m3, i1, i2, i3 = carry
        cv_ref[pl.ds(qb * BQ, BQ), :] = jnp.concatenate([m1, m2, m3], axis=1)
        ci_ref[pl.ds(qb * BQ, BQ), :] = jnp.concatenate([i1, i2, i3], axis=1)
      return 0

    lax.fori_loop(0, M // (2 * BQ), qblock, 0)

    C = cv_ref[...]
    I = ci_ref[...]
    pos = lax.broadcasted_iota(jnp.int32, (M, 384), 1).astype(jnp.float32)
    big = jnp.float32(1e9)
    iv_cols = []
    w_cols = []
    for j in range(K):
        m = jnp.min(C, axis=1, keepdims=True)
        pm = jnp.min(jnp.where(C == m, pos, big), axis=1, keepdims=True)
        iv = jnp.min(jnp.where(pos == pm, I, big), axis=1, keepdims=True)
        iv_cols.append(iv.astype(jnp.int32))
        w_cols.append(1.0 / jnp.clip(m, 1e-16, None))
        if j < K - 1:
            C = jnp.where(pos == pm, inf, C)
    idx_ref[...] = jnp.concatenate(iv_cols, axis=1)
    w_ref[...] = jnp.concatenate(w_cols, axis=1)
    wb_ref[...] = jnp.concatenate(
        [jnp.broadcast_to(w_cols[j], (M, FP)) for j in range(K)], axis=0)


_knn = pl.pallas_call(
    _knn_body,
    in_specs=[
        pl.BlockSpec((M, D), lambda: (0, 0)),
        pl.BlockSpec((D, NPAD), lambda: (0, 0)),
        pl.BlockSpec((NR, XW), lambda: (0, 0)),
    ],
    out_specs=[
        pl.BlockSpec((M, K), lambda: (0, 0)),
        pl.BlockSpec((M, K), lambda: (0, 0)),
        pl.BlockSpec((K * M, FP), lambda: (0, 0)),
        pl.BlockSpec((NR, 8 * FP), lambda: (0, 0)),
    ],
    out_shape=[
        jax.ShapeDtypeStruct((M, K), jnp.int32),
        jax.ShapeDtypeStruct((M, K), jnp.float32),
        jax.ShapeDtypeStruct((K * M, FP), jnp.float32),
        jax.ShapeDtypeStruct((NR, 8 * FP), jnp.float32),
    ],
    scratch_shapes=[
        pltpu.VMEM((M, 3 * 128), jnp.float32),
        pltpu.VMEM((M, 3 * 128), jnp.float32),
    ],
)


def _combine_body(xp_hbm, idx_hbm, wb_hbm, y_hbm, idx_v, rows_v,
                  wb0_v, wb1_v, wb2_v, y_v, sem):
    wid = lax.axis_index("s") * NC + lax.axis_index("c")
    pltpu.sync_copy(idx_hbm.at[wid], idx_v)
    c0 = pltpu.async_copy(xp_hbm.at[idx_v.at[0]], rows_v.at[pl.ds(0, CH)], sem)
    c1 = pltpu.async_copy(xp_hbm.at[idx_v.at[1]], rows_v.at[pl.ds(CH, CH)], sem)
    pltpu.sync_copy(wb_hbm.at[pl.ds(wid * QW, QW)], wb0_v)
    pltpu.sync_copy(wb_hbm.at[pl.ds(M + wid * QW, QW)], wb1_v)
    pltpu.sync_copy(wb_hbm.at[pl.ds(2 * M + wid * QW, QW)], wb2_v)
    c0.wait()
    c1.wait()

    @pl.loop(0, QW)
    def _(q):
        g0 = rows_v[3 * q]
        g1 = rows_v[3 * q + 1]
        g2 = rows_v[3 * q + 2]
        w0 = wb0_v[q]
        w1 = wb1_v[q]
        w2 = wb2_v[q]
        y_v[q] = (g0 * w0 + g1 * w1 + g2 * w2) / ((w0 + w1) + w2)

    pltpu.sync_copy(y_v, y_hbm.at[pl.ds(wid * QW, QW)])


@functools.cache
def _make_combine():
    return pl.kernel(
        _combine_body,
        mesh=plsc.VectorSubcoreMesh(core_axis_name="c", subcore_axis_name="s",
                                    num_cores=NC, num_subcores=NS),
        out_type=jax.ShapeDtypeStruct((M, FP), jnp.float32),
        scratch_types=[
            pltpu.VMEM((2, CH), jnp.int32),
            pltpu.VMEM((RW, FP), jnp.float32),
            pltpu.VMEM((QW, FP), jnp.float32),
            pltpu.VMEM((QW, FP), jnp.float32),
            pltpu.VMEM((QW, FP), jnp.float32),
            pltpu.VMEM((QW, FP), jnp.float32),
            pltpu.SemaphoreType.DMA,
        ],
        compiler_params=pltpu.CompilerParams(use_tc_tiling_on_sc=False),
    )


def kernel(x, pos_x, pos_y, k):
    del k
    pxt = jnp.concatenate(
        [pos_x.T, jnp.full((D, NPAD - N), 1e6, jnp.float32)], axis=1)
    xr = x.reshape(NR, XW)
    idx, w, wb, xp = _knn(pos_y, pxt, xr)
    x_idx = idx.reshape(-1)
    y_idx = jnp.repeat(jnp.arange(M, dtype=jnp.int32), K)
    weights = w.reshape(-1, 1)
    y16 = _make_combine()(xp.reshape(N, FP), x_idx.reshape(NW, 2, CH), wb)
    return (y16[:, :F], x_idx, y_idx, weights)


# speedup vs baseline: 1.0417x; 1.0104x over previous
import functools

import jax
import jax.numpy as jnp
from jax import lax
from jax.experimental import pallas as pl
from jax.experimental.pallas import tpu as pltpu
from jax.experimental.pallas import tpu_sc as plsc

N = 50000
M = 2048
D = 3
F = 3
K = 3
NPAD = 50176
BQ = 16

NR = N // 8
XW = 8 * F

NC = 2
NS = 16
NW = NC * NS
QW = M // NW
RW = QW * K
CH = RW // 2
FP = 16


def _knn_body(py_ref, pxt_ref, xr_ref, idx_ref, w_ref, wb_ref, xp_ref,
              cv_ref, ci_ref):
    xr = xr_ref[...]
    zero13 = jnp.zeros((NR, FP - F), jnp.float32)
    pieces = []
    for j in range(8):
        pieces.append(xr[:, 3 * j:3 * j + 3])
        pieces.append(zero13)
    xp_ref[...] = jnp.concatenate(pieces, axis=1)

    lane = lax.broadcasted_iota(jnp.int32, (1, 128), 1).astype(jnp.float32)
    inf = jnp.float32(jnp.inf)

    def qblock(qb2, _):
      for sub in range(4):
        qb = qb2 * 4 + sub
        py = py_ref[pl.ds(qb * BQ, BQ), :]
        py0 = py[:, 0:1]
        py1 = py[:, 1:2]
        py2 = py[:, 2:3]
        carry = (
            jnp.full((BQ, 128), inf), jnp.full((BQ, 128), inf),
            jnp.full((BQ, 128), inf),
            jnp.zeros((BQ, 128)), jnp.zeros((BQ, 128)), jnp.zeros((BQ, 128)),
        )

        def fold(t, carry):
            m1, m2, m3, i1, i2, i3 = carry
            px = pxt_ref[:, t * 128:(t + 1) * 128]
            c0 = py0 - px[0:1, :]
            c1 = py1 - px[1:2, :]
            c2 = py2 - px[2:3, :]
            c = (c0 * c0 + c1 * c1) + c2 * c2
            icb = jnp.broadcast_to(jnp.float32(t * 128) + lane, (BQ, 128))
            p1 = c < m1
            n_m1 = jnp.minimum(c, m1)
            s1 = jnp.maximum(c, m1)
            n_i1 = jnp.where(p1, icb, i1)
            si1 = jnp.where(p1, i1, icb)
            p2 = s1 < m2
            n_m2 = jnp.minimum(s1, m2)
            s2 = jnp.maximum(s1, m2)
            n_i2 = jnp.where(p2, si1, i2)
            si2 = jnp.where(p2, i2, si1)
            p3 = s2 < m3
            n_m3 = jnp.minimum(s2, m3)
            n_i3 = jnp.where(p3, si2, i3)
            return (n_m1, n_m2, n_m3, n_i1, n_i2, n_i3)

        for t in range(NPAD // 128):
            carry = fold(t, carry)
        m1, m2, m3, i1, i2, i3 = carry
        cv_ref[pl.ds(qb * BQ, BQ), :] = jnp.concatenate([m1, m2, m3], axis=1)
        ci_ref[pl.ds(qb * BQ, BQ), :] = jnp.concatenate([i1, i2, i3], axis=1)
      return 0

    lax.fori_loop(0, M // (4 * BQ), qblock, 0)

    C = cv_ref[...]
    I = ci_ref[...]
    pos = lax.broadcasted_iota(jnp.int32, (M, 384), 1).astype(jnp.float32)
    big = jnp.float32(1e9)
    iv_cols = []
    w_cols = []
    for j in range(K):
        m = jnp.min(C, axis=1, keepdims=True)
        pm = jnp.min(jnp.where(C == m, pos, big), axis=1, keepdims=True)
        iv = jnp.min(jnp.where(pos == pm, I, big), axis=1, keepdims=True)
        iv_cols.append(iv.astype(jnp.int32))
        w_cols.append(1.0 / jnp.clip(m, 1e-16, None))
        if j < K - 1:
            C = jnp.where(pos == pm, inf, C)
    idx_ref[...] = jnp.concatenate(iv_cols, axis=1)
    w_ref[...] = jnp.concatenate(w_cols, axis=1)
    wb_ref[...] = jnp.concatenate(
        [jnp.broadcast_to(w_cols[j], (M, FP)) for j in range(K)], axis=0)


_knn = pl.pallas_call(
    _knn_body,
    in_specs=[
        pl.BlockSpec((M, D), lambda: (0, 0)),
        pl.BlockSpec((D, NPAD), lambda: (0, 0)),
        pl.BlockSpec((NR, XW), lambda: (0, 0)),
    ],
    out_specs=[
        pl.BlockSpec((M, K), lambda: (0, 0)),
        pl.BlockSpec((M, K), lambda: (0, 0)),
        pl.BlockSpec((K * M, FP), lambda: (0, 0)),
        pl.BlockSpec((NR, 8 * FP), lambda: (0, 0)),
    ],
    out_shape=[
        jax.ShapeDtypeStruct((M, K), jnp.int32),
        jax.ShapeDtypeStruct((M, K), jnp.float32),
        jax.ShapeDtypeStruct((K * M, FP), jnp.float32),
        jax.ShapeDtypeStruct((NR, 8 * FP), jnp.float32),
    ],
    scratch_shapes=[
        pltpu.VMEM((M, 3 * 128), jnp.float32),
        pltpu.VMEM((M, 3 * 128), jnp.float32),
    ],
)


def _combine_body(xp_hbm, idx_hbm, wb_hbm, y_hbm, idx_v, rows_v,
                  wb0_v, wb1_v, wb2_v, y_v, sem):
    wid = lax.axis_index("s") * NC + lax.axis_index("c")
    pltpu.sync_copy(idx_hbm.at[wid], idx_v)
    c0 = pltpu.async_copy(xp_hbm.at[idx_v.at[0]], rows_v.at[pl.ds(0, CH)], sem)
    c1 = pltpu.async_copy(xp_hbm.at[idx_v.at[1]], rows_v.at[pl.ds(CH, CH)], sem)
    pltpu.sync_copy(wb_hbm.at[pl.ds(wid * QW, QW)], wb0_v)
    pltpu.sync_copy(wb_hbm.at[pl.ds(M + wid * QW, QW)], wb1_v)
    pltpu.sync_copy(wb_hbm.at[pl.ds(2 * M + wid * QW, QW)], wb2_v)
    c0.wait()
    c1.wait()

    @pl.loop(0, QW)
    def _(q):
        g0 = rows_v[3 * q]
        g1 = rows_v[3 * q + 1]
        g2 = rows_v[3 * q + 2]
        w0 = wb0_v[q]
        w1 = wb1_v[q]
        w2 = wb2_v[q]
        y_v[q] = (g0 * w0 + g1 * w1 + g2 * w2) / ((w0 + w1) + w2)

    pltpu.sync_copy(y_v, y_hbm.at[pl.ds(wid * QW, QW)])


@functools.cache
def _make_combine():
    return pl.kernel(
        _combine_body,
        mesh=plsc.VectorSubcoreMesh(core_axis_name="c", subcore_axis_name="s",
                                    num_cores=NC, num_subcores=NS),
        out_type=jax.ShapeDtypeStruct((M, FP), jnp.float32),
        scratch_types=[
            pltpu.VMEM((2, CH), jnp.int32),
            pltpu.VMEM((RW, FP), jnp.float32),
            pltpu.VMEM((QW, FP), jnp.float32),
            pltpu.VMEM((QW, FP), jnp.float32),
            pltpu.VMEM((QW, FP), jnp.float32),
            pltpu.VMEM((QW, FP), jnp.float32),
            pltpu.SemaphoreType.DMA,
        ],
        compiler_params=pltpu.CompilerParams(use_tc_tiling_on_sc=False),
    )


def kernel(x, pos_x, pos_y, k):
    del k
    pxt = jnp.concatenate(
        [pos_x.T, jnp.full((D, NPAD - N), 1e6, jnp.float32)], axis=1)
    xr = x.reshape(NR, XW)
    idx, w, wb, xp = _knn(pos_y, pxt, xr)
    x_idx = idx.reshape(-1)
    y_idx = jnp.repeat(jnp.arange(M, dtype=jnp.int32), K)
    weights = w.reshape(-1, 1)
    y16 = _make_combine()(xp.reshape(N, FP), x_idx.reshape(NW, 2, CH), wb)
    return (y16[:, :F], x_idx, y_idx, weights)


# speedup vs baseline: 1.0471x; 1.0052x over previous
import functools

import jax
import jax.numpy as jnp
from jax import lax
from jax.experimental import pallas as pl
from jax.experimental.pallas import tpu as pltpu
from jax.experimental.pallas import tpu_sc as plsc

N = 50000
M = 2048
D = 3
F = 3
K = 3
NPAD = 50176
BQ = 16

NR = N // 8
XW = 8 * F

NC = 2
NS = 16
NW = NC * NS
QW = M // NW
RW = QW * K
CH = RW // 2
FP = 16


def _knn_body(py_ref, pxt_ref, xr_ref, idx_ref, w_ref, wb_ref, xp_ref,
              cv_ref, ci_ref):
    xr = xr_ref[...]
    zero13 = jnp.zeros((NR, FP - F), jnp.float32)
    pieces = []
    for j in range(8):
        pieces.append(xr[:, 3 * j:3 * j + 3])
        pieces.append(zero13)
    xp_ref[...] = jnp.concatenate(pieces, axis=1)

    lane = lax.broadcasted_iota(jnp.int32, (1, 128), 1).astype(jnp.float32)
    inf = jnp.float32(jnp.inf)

    def qblock(qb2, _):
      for sub in range(8):
        qb = qb2 * 8 + sub
        py = py_ref[pl.ds(qb * BQ, BQ), :]
        py0 = py[:, 0:1]
        py1 = py[:, 1:2]
        py2 = py[:, 2:3]
        carry = (
            jnp.full((BQ, 128), inf), jnp.full((BQ, 128), inf),
            jnp.full((BQ, 128), inf),
            jnp.zeros((BQ, 128)), jnp.zeros((BQ, 128)), jnp.zeros((BQ, 128)),
        )

        def fold(t, carry):
            m1, m2, m3, i1, i2, i3 = carry
            px = pxt_ref[:, t * 128:(t + 1) * 128]
            c0 = py0 - px[0:1, :]
            c1 = py1 - px[1:2, :]
            c2 = py2 - px[2:3, :]
            c = (c0 * c0 + c1 * c1) + c2 * c2
            icb = jnp.broadcast_to(jnp.float32(t * 128) + lane, (BQ, 128))
            p1 = c < m1
            n_m1 = jnp.minimum(c, m1)
            s1 = jnp.maximum(c, m1)
            n_i1 = jnp.where(p1, icb, i1)
            si1 = jnp.where(p1, i1, icb)
            p2 = s1 < m2
            n_m2 = jnp.minimum(s1, m2)
            s2 = jnp.maximum(s1, m2)
            n_i2 = jnp.where(p2, si1, i2)
            si2 = jnp.where(p2, i2, si1)
            p3 = s2 < m3
            n_m3 = jnp.minimum(s2, m3)
            n_i3 = jnp.where(p3, si2, i3)
            return (n_m1, n_m2, n_m3, n_i1, n_i2, n_i3)

        for t in range(NPAD // 128):
            carry = fold(t, carry)
        m1, m2, m3, i1, i2, i3 = carry
        cv_ref[pl.ds(qb * BQ, BQ), :] = jnp.concatenate([m1, m2, m3], axis=1)
        ci_ref[pl.ds(qb * BQ, BQ), :] = jnp.concatenate([i1, i2, i3], axis=1)
      return 0

    lax.fori_loop(0, M // (8 * BQ), qblock, 0)

    C = cv_ref[...]
    I = ci_ref[...]
    pos = lax.broadcasted_iota(jnp.int32, (M, 384), 1).astype(jnp.float32)
    big = jnp.float32(1e9)
    iv_cols = []
    w_cols = []
    for j in range(K):
        m = jnp.min(C, axis=1, keepdims=True)
        pm = jnp.min(jnp.where(C == m, pos, big), axis=1, keepdims=True)
        iv = jnp.min(jnp.where(pos == pm, I, big), axis=1, keepdims=True)
        iv_cols.append(iv.astype(jnp.int32))
        w_cols.append(1.0 / jnp.clip(m, 1e-16, None))
        if j < K - 1:
            C = jnp.where(pos == pm, inf, C)
    idx_ref[...] = jnp.concatenate(iv_cols, axis=1)
    w_ref[...] = jnp.concatenate(w_cols, axis=1)
    wb_ref[...] = jnp.concatenate(
        [jnp.broadcast_to(w_cols[j], (M, FP)) for j in range(K)], axis=0)


_knn = pl.pallas_call(
    _knn_body,
    in_specs=[
        pl.BlockSpec((M, D), lambda: (0, 0)),
        pl.BlockSpec((D, NPAD), lambda: (0, 0)),
        pl.BlockSpec((NR, XW), lambda: (0, 0)),
    ],
    out_specs=[
        pl.BlockSpec((M, K), lambda: (0, 0)),
        pl.BlockSpec((M, K), lambda: (0, 0)),
        pl.BlockSpec((K * M, FP), lambda: (0, 0)),
        pl.BlockSpec((NR, 8 * FP), lambda: (0, 0)),
    ],
    out_shape=[
        jax.ShapeDtypeStruct((M, K), jnp.int32),
        jax.ShapeDtypeStruct((M, K), jnp.float32),
        jax.ShapeDtypeStruct((K * M, FP), jnp.float32),
        jax.ShapeDtypeStruct((NR, 8 * FP), jnp.float32),
    ],
    scratch_shapes=[
        pltpu.VMEM((M, 3 * 128), jnp.float32),
        pltpu.VMEM((M, 3 * 128), jnp.float32),
    ],
)


def _combine_body(xp_hbm, idx_hbm, wb_hbm, y_hbm, idx_v, rows_v,
                  wb0_v, wb1_v, wb2_v, y_v, sem):
    wid = lax.axis_index("s") * NC + lax.axis_index("c")
    pltpu.sync_copy(idx_hbm.at[wid], idx_v)
    c0 = pltpu.async_copy(xp_hbm.at[idx_v.at[0]], rows_v.at[pl.ds(0, CH)], sem)
    c1 = pltpu.async_copy(xp_hbm.at[idx_v.at[1]], rows_v.at[pl.ds(CH, CH)], sem)
    pltpu.sync_copy(wb_hbm.at[pl.ds(wid * QW, QW)], wb0_v)
    pltpu.sync_copy(wb_hbm.at[pl.ds(M + wid * QW, QW)], wb1_v)
    pltpu.sync_copy(wb_hbm.at[pl.ds(2 * M + wid * QW, QW)], wb2_v)
    c0.wait()
    c1.wait()

    @pl.loop(0, QW)
    def _(q):
        g0 = rows_v[3 * q]
        g1 = rows_v[3 * q + 1]
        g2 = rows_v[3 * q + 2]
        w0 = wb0_v[q]
        w1 = wb1_v[q]
        w2 = wb2_v[q]
        y_v[q] = (g0 * w0 + g1 * w1 + g2 * w2) / ((w0 + w1) + w2)

    pltpu.sync_copy(y_v, y_hbm.at[pl.ds(wid * QW, QW)])


@functools.cache
def _make_combine():
    return pl.kernel(
        _combine_body,
        mesh=plsc.VectorSubcoreMesh(core_axis_name="c", subcore_axis_name="s",
                                    num_cores=NC, num_subcores=NS),
        out_type=jax.ShapeDtypeStruct((M, FP), jnp.float32),
        scratch_types=[
            pltpu.VMEM((2, CH), jnp.int32),
            pltpu.VMEM((RW, FP), jnp.float32),
            pltpu.VMEM((QW, FP), jnp.float32),
            pltpu.VMEM((QW, FP), jnp.float32),
            pltpu.VMEM((QW, FP), jnp.float32),
            pltpu.VMEM((QW, FP), jnp.float32),
            pltpu.SemaphoreType.DMA,
        ],
        compiler_params=pltpu.CompilerParams(use_tc_tiling_on_sc=False),
    )


def kernel(x, pos_x, pos_y, k):
    del k
    pxt = jnp.concatenate(
        [pos_x.T, jnp.full((D, NPAD - N), 1e6, jnp.float32)], axis=1)
    xr = x.reshape(NR, XW)
    idx, w, wb, xp = _knn(pos_y, pxt, xr)
    x_idx = idx.reshape(-1)
    y_idx = jnp.repeat(jnp.arange(M, dtype=jnp.int32), K)
    weights = w.reshape(-1, 1)
    y16 = _make_combine()(xp.reshape(N, FP), x_idx.reshape(NW, 2, CH), wb)
    return (y16[:, :F], x_idx, y_idx, weights)
